# baseline (speedup 1.0000x reference)
import functools

import jax
import jax.numpy as jnp
from jax import lax
from jax.experimental import pallas as pl
from jax.experimental.pallas import tpu as pltpu
from jax.experimental.pallas import tpu_sc as plsc

B = 4096
L = 50
LQ = 5
D = 64
DENSE = 128

_NC = 2
_NS = 16
_NW = _NC * _NS
_CH = 128


_NBUF = 2


def pair_chunks(v, n_chunks):
  c = (v.reshape(-1, _CH) >> 1).astype(jnp.int32)
  pad = n_chunks - c.shape[0]
  return jnp.concatenate([c, jnp.zeros((pad, _CH), jnp.int32)])


def _sc_gather_multi(table, idxs):
  d = table.shape[1]
  seg_ch = []
  for ix in idxs:
    n_ch = ix.shape[0] // _NW
    assert n_ch * _NW == ix.shape[0] and n_ch % 8 == 0
    seg_ch.append(n_ch)
  tot_ch = sum(seg_ch)

  mesh = plsc.VectorSubcoreMesh(core_axis_name="c", subcore_axis_name="s")

  @functools.partial(
      pl.kernel,
      mesh=mesh,
      out_type=tuple(
          jax.ShapeDtypeStruct((ix.shape[0] * _CH, d), jnp.float32)
          for ix in idxs),
      scratch_types=[
          pltpu.VMEM((tot_ch, _CH), jnp.int32),
          pltpu.VMEM((_NBUF, _CH, d), jnp.float32),
      ] + [pltpu.SemaphoreType.DMA] * _NBUF,
  )
  def k(table_hbm, *refs):
    idx_hbms = refs[:len(idxs)]
    out_hbms = refs[len(idxs):2 * len(idxs)]
    idx_v = refs[2 * len(idxs)]
    rows_v = refs[2 * len(idxs) + 1]
    sems = refs[2 * len(idxs) + 2:]
    wid = lax.axis_index("s") * _NC + lax.axis_index("c")

    soff = 0
    for s, n_ch in enumerate(seg_ch):
      pltpu.sync_copy(idx_hbms[s].at[pl.ds(wid * n_ch, n_ch)],
                      idx_v.at[pl.ds(soff, n_ch)])
      soff += n_ch

    soff = 0
    for s, n_ch in enumerate(seg_ch):
      out = out_hbms[s]
      rbase = wid * n_ch
      n_grp = n_ch // _NBUF

      def body(g, carry, soff=soff, out=out, rbase=rbase):
        handles = []
        for b in range(_NBUF):
          i = g * _NBUF + b
          handles.append(
              pltpu.async_copy(table_hbm.at[idx_v.at[soff + i]],
                               rows_v.at[b], sems[b]))
        for b in range(_NBUF):
          i = g * _NBUF + b
          handles[b].wait()
          pltpu.sync_copy(rows_v.at[b],
                          out.at[pl.ds((rbase + i) * _CH, _CH)])
        return carry

      if n_grp > 0:
        lax.fori_loop(0, n_grp, body, 0, unroll=False)
      for i in range(n_grp * _NBUF, n_ch):
        pltpu.async_copy(table_hbm.at[idx_v.at[soff + i]], rows_v.at[0],
                         sems[0]).wait()
        pltpu.sync_copy(rows_v.at[0], out.at[pl.ds((rbase + i) * _CH, _CH)])
      soff += n_ch

  return k(table, *idxs)


def _dense_body(
    s_raw, c_raw, b_raw, it_raw, iq_raw,
    src_i, clk_i, brw_i, iq_i, it_i, lbl,
    Wti, bti, Wtq, btq,
    Wsq, bsq, qsq, Wsc, bsc, qsc, Wbi, bbi, qbi,
    Wir, bir, Wur, bur, Wua, bua, qua,
    Wiv1, biv1, Wiv2, biv2,
    WuA1, buA1, WuA2, buA2, WiA1, biA1, WiA2, biA2,
    o_bce, o_s1, o_s1i,
):
  pid = pl.program_id(0)
  bb = src_i.shape[0]
  inv_b = jnp.float32(1.0 / B)

  def half(x128, idx, ll):
    x3 = x128.reshape(bb, ll, 2 * D)
    p3 = (idx & 1)[:, :, None]
    return jnp.where(p3 == 1, x3[:, :, D:2 * D], x3[:, :, 0:D])

  def attn_pool(x2, idx, ll, Wt, bt, W, b, q):
    A = jnp.dot(Wt[...], W[...], preferred_element_type=jnp.float32)
    c = jnp.dot(bt[...], W[...], preferred_element_type=jnp.float32) + b[...]
    h = jnp.tanh(jnp.dot(x2, A, preferred_element_type=jnp.float32) + c)
    s = jnp.dot(h, q[...].reshape(DENSE, 1),
                preferred_element_type=jnp.float32).reshape(bb, ll)
    s = jnp.where(idx == 0, jnp.float32(-1e9), s)
    a = jax.nn.softmax(s, axis=-1)
    pooled = jnp.sum(a[:, :, None] * x2.reshape(bb, ll, D), axis=1)
    return jnp.dot(pooled, Wt[...], preferred_element_type=jnp.float32) + bt[...]

  def iv_pool(x2, idx, ll):
    m = (idx != 0).astype(jnp.float32)
    pooled = jnp.sum(m[:, :, None] * x2.reshape(bb, ll, D), axis=1)
    cnt = jnp.maximum(jnp.sum(m, axis=1, keepdims=True), 1.0)
    pooled = pooled / cnt
    h = jnp.tanh(jnp.dot(pooled, Wiv1[...],
                         preferred_element_type=jnp.float32) + biv1[...])
    return jnp.tanh(jnp.dot(h, Wiv2[...],
                            preferred_element_type=jnp.float32) + biv2[...])

  def fc_sig(x, W1, b1, W2, b2):
    h = jax.nn.relu(jnp.dot(x, W1[...],
                            preferred_element_type=jnp.float32) + b1[...])
    lg = jnp.sum(h * W2[...], axis=-1, keepdims=True) + b2[...]
    return jax.nn.sigmoid(lg)

  it64 = jnp.where((it_i[...] & 1) == 1, it_raw[..., D:2 * D],
                   it_raw[..., 0:D])
  s64 = half(s_raw[...], src_i[...], L).reshape(bb * L, D)
  c64 = half(c_raw[...], clk_i[...], L).reshape(bb * L, D)
  b64 = half(b_raw[...], brw_i[...], L).reshape(bb * L, D)
  iq64 = half(iq_raw[...], iq_i[...], LQ).reshape(bb * LQ, D)

  item_emb = jnp.dot(it64, Wti[...],
                     preferred_element_type=jnp.float32) + bti[...]
  query_rep = attn_pool(s64, src_i[...], L, Wtq, btq, Wsq, bsq, qsq)
  click_rep = attn_pool(c64, clk_i[...], L, Wti, bti, Wsc, bsc, qsc)
  browse_rep = attn_pool(b64, brw_i[...], L, Wti, bti, Wbi, bbi, qbi)

  iv_feats = iv_pool(s64, src_i[...], L)
  d1 = iv_feats - browse_rep
  s1_part = jnp.sum(d1 * d1) * (inv_b / D)

  uw = fc_sig(jnp.concatenate([iv_feats, browse_rep], axis=-1),
              WuA1, buA1, WuA2, buA2)
  iv_user = uw * iv_feats + (1.0 - uw) * browse_rep

  def u_branch(x):
    u = jnp.tanh(jnp.dot(x, Wur[...], preferred_element_type=jnp.float32)
                 + bur[...])
    hu = jnp.tanh(jnp.dot(u, Wua[...], preferred_element_type=jnp.float32)
                  + bua[...])
    su = jnp.dot(hu, qua[...].reshape(100, 1),
                 preferred_element_type=jnp.float32)
    return u, su

  u0, su0 = u_branch(iv_user)
  u1, su1 = u_branch(query_rep)
  u2, su2 = u_branch(click_rep)
  su = jnp.concatenate([su0, su1, su2], axis=-1)
  au = jax.nn.softmax(su, axis=-1)
  user_rep = (au[:, 0:1] * u0 + au[:, 1:2] * u1 + au[:, 2:3] * u2)

  iv_item = iv_pool(iq64, iq_i[...], LQ)
  d2 = iv_item - item_emb
  s1i_part = jnp.sum(d2 * d2) * (inv_b / D)

  iw = fc_sig(jnp.concatenate([iv_item, item_emb], axis=-1),
              WiA1, biA1, WiA2, biA2)
  item_rep0 = iw * iv_item + (1.0 - iw) * item_emb
  item_rep = jnp.tanh(jnp.dot(item_rep0, Wir[...],
                              preferred_element_type=jnp.float32) + bir[...])

  logits = jnp.sum(item_rep * user_rep, axis=-1, keepdims=True)
  prob = jnp.clip(jax.nn.sigmoid(logits), 1e-7, 1.0 - 1e-7)
  y = lbl[...]
  bce_part = jnp.sum(-(y * jnp.log(prob) + (1.0 - y) * jnp.log(1.0 - prob))
                     ) * inv_b

  @pl.when(pid == 0)
  def _():
    o_bce[...] = jnp.zeros_like(o_bce)
    o_s1[...] = jnp.zeros_like(o_s1)
    o_s1i[...] = jnp.zeros_like(o_s1i)

  o_bce[...] += bce_part
  o_s1[...] += s1_part
  o_s1i[...] += s1i_part


def _dense(interpret, *args):
  bb = 128
  grid = B // bb

  def full(x):
    return pl.BlockSpec(x.shape, lambda i: (0,) * x.ndim)

  def rows(x, mult):
    return pl.BlockSpec((bb * mult,) + x.shape[1:],
                        lambda i: (i,) + (0,) * (x.ndim - 1))

  mults = (L, L, L, 1, LQ, 1, 1, 1, 1, 1, 1)
  weights = args[11:]
  in_specs = [rows(a, m) for a, m in zip(args[:11], mults)
              ] + [full(w) for w in weights]
  out_spec = pl.BlockSpec((1, 1), lambda i: (0, 0))
  return pl.pallas_call(
      _dense_body,
      grid=(grid,),
      in_specs=in_specs,
      out_specs=(out_spec, out_spec, out_spec),
      out_shape=tuple(jax.ShapeDtypeStruct((1, 1), jnp.float32)
                      for _ in range(3)),
      interpret=interpret,
  )(*args)


def kernel(browse_item, src_qry, search_click, item, item_qry, labels,
           item_table, qry_table, Wti, bti, Wtq, btq, Wsq, bsq, qsq,
           Wsc, bsc, qsc, Wbi, bbi, qbi, Wir, bir, Wur, bur, Wua, bua, qua,
           Wiv1, biv1, Wiv2, biv2, WuA1, buA1, WuA2, buA2,
           WiA1, biA1, WiA2, biA2):
  b_raw, c_raw, it_raw = _sc_gather_multi(
      item_table.reshape(-1, 2 * D),
      [pair_chunks(browse_item, 1792),
       pair_chunks(search_click, 1792), pair_chunks(item, 256)])
  s_raw, iq_raw = _sc_gather_multi(
      qry_table.reshape(-1, 2 * D),
      [pair_chunks(src_qry, 1792), pair_chunks(item_qry, 256)])

  r1 = lambda v: v.reshape(1, -1)
  o_bce, o_s1, o_s1i = _dense(
      False,
      s_raw, c_raw, b_raw, it_raw, iq_raw,
      src_qry, search_click, browse_item, item_qry, item.reshape(B, 1),
      labels.reshape(B, 1),
      Wti, r1(bti), Wtq, r1(btq),
      Wsq, r1(bsq), r1(qsq), Wsc, r1(bsc), r1(qsc), Wbi, r1(bbi), r1(qbi),
      Wir, r1(bir), Wur, r1(bur), Wua, r1(bua), r1(qua),
      Wiv1, r1(biv1), Wiv2, r1(biv2),
      r1(WuA1) if WuA1.ndim == 1 else WuA1, r1(buA1),
      WuA2.reshape(1, -1), r1(buA2),
      WiA1, r1(biA1), WiA2.reshape(1, -1), r1(biA2),
  )
  return (o_bce[0, 0], o_s1[0, 0], o_s1i[0, 0])


# speedup vs baseline: 4.0992x; 4.0992x over previous
import functools

import jax
import jax.numpy as jnp
from jax import lax
from jax.experimental import pallas as pl
from jax.experimental.pallas import tpu as pltpu
from jax.experimental.pallas import tpu_sc as plsc

B = 4096
L = 50
LQ = 5
D = 64
DENSE = 128

_NC = 2
_NS = 16
_NW = _NC * _NS
_CH = 128


_NBUF = 2


def _sc_gather_multi(table, idxs):
  d = table.shape[1]
  seg_ch = []
  for ix in idxs:
    n_ch = ix.shape[0] // _NW
    assert n_ch * _NW == ix.shape[0]
    seg_ch.append(n_ch)
  tot_ch = sum(seg_ch)

  mesh = plsc.VectorSubcoreMesh(core_axis_name="c", subcore_axis_name="s")

  @functools.partial(
      pl.kernel,
      mesh=mesh,
      out_type=tuple(
          jax.ShapeDtypeStruct((ix.shape[0] * _CH, d), jnp.float32)
          for ix in idxs),
      scratch_types=[
          pltpu.VMEM((tot_ch, _CH), jnp.int32),
          pltpu.VMEM((_NBUF, _CH, d), jnp.float32),
      ] + [pltpu.SemaphoreType.DMA] * _NBUF,
      compiler_params=pltpu.CompilerParams(use_tc_tiling_on_sc=False),
  )
  def k(table_hbm, *refs):
    idx_hbms = refs[:len(idxs)]
    out_hbms = refs[len(idxs):2 * len(idxs)]
    idx_v = refs[2 * len(idxs)]
    rows_v = refs[2 * len(idxs) + 1]
    sems = refs[2 * len(idxs) + 2:]
    wid = lax.axis_index("s") * _NC + lax.axis_index("c")

    soff = 0
    for s, n_ch in enumerate(seg_ch):
      pltpu.sync_copy(idx_hbms[s].at[pl.ds(wid * n_ch, n_ch)],
                      idx_v.at[pl.ds(soff, n_ch)])
      soff += n_ch

    soff = 0
    for s, n_ch in enumerate(seg_ch):
      out = out_hbms[s]
      rbase = wid * n_ch
      n_grp = n_ch // _NBUF

      def body(g, carry, soff=soff, out=out, rbase=rbase):
        handles = []
        for b in range(_NBUF):
          i = g * _NBUF + b
          handles.append(
              pltpu.async_copy(table_hbm.at[idx_v.at[soff + i]],
                               rows_v.at[b], sems[b]))
        for b in range(_NBUF):
          i = g * _NBUF + b
          handles[b].wait()
          pltpu.sync_copy(rows_v.at[b],
                          out.at[pl.ds((rbase + i) * _CH, _CH)])
        return carry

      if n_grp > 0:
        lax.fori_loop(0, n_grp, body, 0, unroll=False)
      for i in range(n_grp * _NBUF, n_ch):
        pltpu.async_copy(table_hbm.at[idx_v.at[soff + i]], rows_v.at[0],
                         sems[0]).wait()
        pltpu.sync_copy(rows_v.at[0], out.at[pl.ds((rbase + i) * _CH, _CH)])
      soff += n_ch

  return k(table, *idxs)


def _dense_body(
    s_raw, c_raw, b_raw, it_raw, iq_raw,
    src_i, clk_i, brw_i, iq_i, lbl,
    Wti, bti, Wtq, btq,
    Wsq, bsq, qsq, Wsc, bsc, qsc, Wbi, bbi, qbi,
    Wir, bir, Wur, bur, Wua, bua, qua,
    Wiv1, biv1, Wiv2, biv2,
    WuA1, buA1, WuA2, buA2, WiA1, biA1, WiA2, biA2,
    o_bce, o_s1, o_s1i,
):
  pid = pl.program_id(0)
  bb = src_i.shape[0]
  inv_b = jnp.float32(1.0 / B)

  def attn_pool(xp, idxp, Wt, bt, W, b, q):
    lh = L // 2
    A = jnp.dot(Wt[...], W[...], preferred_element_type=jnp.float32)
    c = jnp.dot(bt[...], W[...], preferred_element_type=jnp.float32) + b[...]
    xe = xp[:, 0:D]
    xo = xp[:, D:2 * D]
    qc = q[...].reshape(DENSE, 1)
    he = jnp.tanh(jnp.dot(xe, A, preferred_element_type=jnp.float32) + c)
    ho = jnp.tanh(jnp.dot(xo, A, preferred_element_type=jnp.float32) + c)
    se = jnp.dot(he, qc, preferred_element_type=jnp.float32).reshape(bb, lh)
    so = jnp.dot(ho, qc, preferred_element_type=jnp.float32).reshape(bb, lh)
    s = jnp.concatenate([se, so], axis=-1)
    s = jnp.where(idxp == 0, jnp.float32(-1e9), s)
    a = jax.nn.softmax(s, axis=-1)
    pooled = (jnp.sum(a[:, 0:lh, None] * xe.reshape(bb, lh, D), axis=1)
              + jnp.sum(a[:, lh:L, None] * xo.reshape(bb, lh, D), axis=1))
    return jnp.dot(pooled, Wt[...], preferred_element_type=jnp.float32) + bt[...]

  def iv_pool_pair(xp, idxp):
    lh = L // 2
    m = (idxp != 0).astype(jnp.float32)
    xe = xp[:, 0:D]
    xo = xp[:, D:2 * D]
    pooled = (jnp.sum(m[:, 0:lh, None] * xe.reshape(bb, lh, D), axis=1)
              + jnp.sum(m[:, lh:L, None] * xo.reshape(bb, lh, D), axis=1))
    cnt = jnp.maximum(jnp.sum(m, axis=1, keepdims=True), 1.0)
    pooled = pooled / cnt
    h = jnp.tanh(jnp.dot(pooled, Wiv1[...],
                         preferred_element_type=jnp.float32) + biv1[...])
    return jnp.tanh(jnp.dot(h, Wiv2[...],
                            preferred_element_type=jnp.float32) + biv2[...])

  def iv_pool(x2, idx, ll):
    m = (idx != 0).astype(jnp.float32)
    pooled = jnp.sum(m[:, :, None] * x2.reshape(bb, ll, D), axis=1)
    cnt = jnp.maximum(jnp.sum(m, axis=1, keepdims=True), 1.0)
    pooled = pooled / cnt
    h = jnp.tanh(jnp.dot(pooled, Wiv1[...],
                         preferred_element_type=jnp.float32) + biv1[...])
    return jnp.tanh(jnp.dot(h, Wiv2[...],
                            preferred_element_type=jnp.float32) + biv2[...])

  def fc_sig(x, W1, b1, W2, b2):
    h = jax.nn.relu(jnp.dot(x, W1[...],
                            preferred_element_type=jnp.float32) + b1[...])
    lg = jnp.sum(h * W2[...], axis=-1, keepdims=True) + b2[...]
    return jax.nn.sigmoid(lg)

  item_emb = jnp.dot(it_raw[...], Wti[...],
                     preferred_element_type=jnp.float32) + bti[...]
  query_rep = attn_pool(s_raw[...], src_i[...], Wtq, btq, Wsq, bsq, qsq)
  click_rep = attn_pool(c_raw[...], clk_i[...], Wti, bti, Wsc, bsc, qsc)
  browse_rep = attn_pool(b_raw[...], brw_i[...], Wti, bti, Wbi, bbi, qbi)

  iv_feats = iv_pool_pair(s_raw[...], src_i[...])
  d1 = iv_feats - browse_rep
  s1_part = jnp.sum(d1 * d1) * (inv_b / D)

  uw = fc_sig(jnp.concatenate([iv_feats, browse_rep], axis=-1),
              WuA1, buA1, WuA2, buA2)
  iv_user = uw * iv_feats + (1.0 - uw) * browse_rep

  def u_branch(x):
    u = jnp.tanh(jnp.dot(x, Wur[...], preferred_element_type=jnp.float32)
                 + bur[...])
    hu = jnp.tanh(jnp.dot(u, Wua[...], preferred_element_type=jnp.float32)
                  + bua[...])
    su = jnp.dot(hu, qua[...].reshape(100, 1),
                 preferred_element_type=jnp.float32)
    return u, su

  u0, su0 = u_branch(iv_user)
  u1, su1 = u_branch(query_rep)
  u2, su2 = u_branch(click_rep)
  su = jnp.concatenate([su0, su1, su2], axis=-1)
  au = jax.nn.softmax(su, axis=-1)
  user_rep = (au[:, 0:1] * u0 + au[:, 1:2] * u1 + au[:, 2:3] * u2)

  iv_item = iv_pool(iq_raw[...], iq_i[...], LQ)
  d2 = iv_item - item_emb
  s1i_part = jnp.sum(d2 * d2) * (inv_b / D)

  iw = fc_sig(jnp.concatenate([iv_item, item_emb], axis=-1),
              WiA1, biA1, WiA2, biA2)
  item_rep0 = iw * iv_item + (1.0 - iw) * item_emb
  item_rep = jnp.tanh(jnp.dot(item_rep0, Wir[...],
                              preferred_element_type=jnp.float32) + bir[...])

  logits = jnp.sum(item_rep * user_rep, axis=-1, keepdims=True)
  prob = jnp.clip(jax.nn.sigmoid(logits), 1e-7, 1.0 - 1e-7)
  y = lbl[...]
  bce_part = jnp.sum(-(y * jnp.log(prob) + (1.0 - y) * jnp.log(1.0 - prob))
                     ) * inv_b

  @pl.when(pid == 0)
  def _():
    o_bce[...] = jnp.zeros_like(o_bce)
    o_s1[...] = jnp.zeros_like(o_s1)
    o_s1i[...] = jnp.zeros_like(o_s1i)

  o_bce[...] += bce_part
  o_s1[...] += s1_part
  o_s1i[...] += s1i_part


def _dense(interpret, *args):
  bb = 128
  grid = B // bb

  def full(x):
    return pl.BlockSpec(x.shape, lambda i: (0,) * x.ndim)

  def rows(x):
    blk = x.shape[0] // grid
    return pl.BlockSpec((blk,) + x.shape[1:],
                        lambda i: (i,) + (0,) * (x.ndim - 1))

  weights = args[10:]
  in_specs = [rows(a) for a in args[:10]] + [full(w) for w in weights]
  out_spec = pl.BlockSpec((1, 1), lambda i: (0, 0))
  return pl.pallas_call(
      _dense_body,
      grid=(grid,),
      in_specs=in_specs,
      out_specs=(out_spec, out_spec, out_spec),
      out_shape=tuple(jax.ShapeDtypeStruct((1, 1), jnp.float32)
                      for _ in range(3)),
      interpret=interpret,
  )(*args)


def kernel(browse_item, src_qry, search_click, item, item_qry, labels,
           item_table, qry_table, Wti, bti, Wtq, btq, Wsq, bsq, qsq,
           Wsc, bsc, qsc, Wbi, bbi, qbi, Wir, bir, Wur, bur, Wua, bua, qua,
           Wiv1, biv1, Wiv2, biv2, WuA1, buA1, WuA2, buA2,
           WiA1, biA1, WiA2, biA2):
  b_raw, c_raw, it_raw = _sc_gather_multi(
      item_table,
      [browse_item.reshape(-1, _CH), search_click.reshape(-1, _CH),
       item.reshape(-1, _CH)])
  s_raw, iq_raw = _sc_gather_multi(
      qry_table,
      [src_qry.reshape(-1, _CH), item_qry.reshape(-1, _CH)])

  pair = lambda x: x.reshape(-1, 2 * D)
  lperm = lambda ix: jnp.concatenate([ix[:, 0::2], ix[:, 1::2]], axis=-1)

  r1 = lambda v: v.reshape(1, -1)
  o_bce, o_s1, o_s1i = _dense(
      False,
      pair(s_raw), pair(c_raw), pair(b_raw), it_raw, iq_raw,
      lperm(src_qry), lperm(search_click), lperm(browse_item), item_qry,
      labels.reshape(B, 1),
      Wti, r1(bti), Wtq, r1(btq),
      Wsq, r1(bsq), r1(qsq), Wsc, r1(bsc), r1(qsc), Wbi, r1(bbi), r1(qbi),
      Wir, r1(bir), Wur, r1(bur), Wua, r1(bua), r1(qua),
      Wiv1, r1(biv1), Wiv2, r1(biv2),
      r1(WuA1) if WuA1.ndim == 1 else WuA1, r1(buA1),
      WuA2.reshape(1, -1), r1(buA2),
      WiA1, r1(biA1), WiA2.reshape(1, -1), r1(biA2),
  )
  return (o_bce[0, 0], o_s1[0, 0], o_s1i[0, 0])


# speedup vs baseline: 4.2136x; 1.0279x over previous
import functools

import jax
import jax.numpy as jnp
from jax import lax
from jax.experimental import pallas as pl
from jax.experimental.pallas import tpu as pltpu
from jax.experimental.pallas import tpu_sc as plsc

B = 4096
L = 50
LQ = 5
D = 64
DENSE = 128

_NC = 2
_NS = 16
_NW = _NC * _NS
_CH = 128


_NBUF = 2


def _sc_gather_multi(table, idxs):
  d = table.shape[1]
  seg_ch = []
  tot_ch = 0
  for ix in idxs:
    n_ch = ix.shape[0] // _NW
    if n_ch * _NW == ix.shape[0]:
      seg_ch.append(n_ch)
      tot_ch += n_ch
    else:
      seg_ch.append(None)
      tot_ch += 1

  mesh = plsc.VectorSubcoreMesh(core_axis_name="c", subcore_axis_name="s")

  @functools.partial(
      pl.kernel,
      mesh=mesh,
      out_type=tuple(
          jax.ShapeDtypeStruct((ix.shape[0] * _CH, d), jnp.float32)
          for ix in idxs),
      scratch_types=[
          pltpu.VMEM((tot_ch, _CH), jnp.int32),
          pltpu.VMEM((_NBUF, _CH, d), jnp.float32),
      ] + [pltpu.SemaphoreType.DMA] * _NBUF,
      compiler_params=pltpu.CompilerParams(use_tc_tiling_on_sc=False),
  )
  def k(table_hbm, *refs):
    idx_hbms = refs[:len(idxs)]
    out_hbms = refs[len(idxs):2 * len(idxs)]
    idx_v = refs[2 * len(idxs)]
    rows_v = refs[2 * len(idxs) + 1]
    sems = refs[2 * len(idxs) + 2:]
    wid = lax.axis_index("s") * _NC + lax.axis_index("c")

    soff = 0
    for s, n_ch in enumerate(seg_ch):
      if n_ch is None:
        soff += 1
        continue
      pltpu.sync_copy(idx_hbms[s].at[pl.ds(wid * n_ch, n_ch)],
                      idx_v.at[pl.ds(soff, n_ch)])
      soff += n_ch

    soff = 0
    for s, n_ch in enumerate(seg_ch):
      out = out_hbms[s]
      if n_ch is None:
        nck = idxs[s].shape[0]
        q, r = divmod(nck, _NW)
        base = wid * q + jnp.minimum(wid, r)
        cnt = q + (wid < r).astype(jnp.int32)

        def dbody(i, carry, soff=soff, out=out, base=base, ih=idx_hbms[s]):
          pltpu.sync_copy(ih.at[pl.ds(base + i, 1)],
                          idx_v.at[pl.ds(soff, 1)])
          pltpu.async_copy(table_hbm.at[idx_v.at[soff]], rows_v.at[0],
                           sems[0]).wait()
          pltpu.sync_copy(rows_v.at[0],
                          out.at[pl.ds((base + i) * _CH, _CH)])
          return carry

        lax.fori_loop(0, cnt, dbody, 0, unroll=False)
        soff += 1
        continue
      rbase = wid * n_ch
      n_grp = n_ch // _NBUF

      def body(g, carry, soff=soff, out=out, rbase=rbase):
        handles = []
        for b in range(_NBUF):
          i = g * _NBUF + b
          handles.append(
              pltpu.async_copy(table_hbm.at[idx_v.at[soff + i]],
                               rows_v.at[b], sems[b]))
        for b in range(_NBUF):
          i = g * _NBUF + b
          handles[b].wait()
          pltpu.sync_copy(rows_v.at[b],
                          out.at[pl.ds((rbase + i) * _CH, _CH)])
        return carry

      if n_grp > 0:
        lax.fori_loop(0, n_grp, body, 0, unroll=False)
      for i in range(n_grp * _NBUF, n_ch):
        pltpu.async_copy(table_hbm.at[idx_v.at[soff + i]], rows_v.at[0],
                         sems[0]).wait()
        pltpu.sync_copy(rows_v.at[0], out.at[pl.ds((rbase + i) * _CH, _CH)])
      soff += n_ch

  return k(table, *idxs)


def _dense_body(
    s_raw, c_raw, b_raw, it_raw, iq_raw,
    src_i, clk_i, brw_i, iq_i, lbl,
    Wti, bti, Wtq, btq,
    Wsq, bsq, qsq, Wsc, bsc, qsc, Wbi, bbi, qbi,
    Wir, bir, Wur, bur, Wua, bua, qua,
    Wiv1, biv1, Wiv2, biv2,
    WuA1, buA1, WuA2, buA2, WiA1, biA1, WiA2, biA2,
    o_bce, o_s1, o_s1i,
):
  pid = pl.program_id(0)
  bb = src_i.shape[0]
  inv_b = jnp.float32(1.0 / B)

  def attn_pool(x2, idx, ll, Wt, bt, W, b, q):
    A = jnp.dot(Wt[...], W[...], preferred_element_type=jnp.float32)
    c = jnp.dot(bt[...], W[...], preferred_element_type=jnp.float32) + b[...]
    h = jnp.tanh(jnp.dot(x2, A, preferred_element_type=jnp.float32) + c)
    s = jnp.dot(h, q[...].reshape(DENSE, 1),
                preferred_element_type=jnp.float32).reshape(bb, ll)
    s = jnp.where(idx == 0, jnp.float32(-1e9), s)
    a = jax.nn.softmax(s, axis=-1)
    pooled = jnp.sum(a[:, :, None] * x2.reshape(bb, ll, D), axis=1)
    return jnp.dot(pooled, Wt[...], preferred_element_type=jnp.float32) + bt[...]

  def iv_pool(x2, idx, ll):
    m = (idx != 0).astype(jnp.float32)
    pooled = jnp.sum(m[:, :, None] * x2.reshape(bb, ll, D), axis=1)
    cnt = jnp.maximum(jnp.sum(m, axis=1, keepdims=True), 1.0)
    pooled = pooled / cnt
    h = jnp.tanh(jnp.dot(pooled, Wiv1[...],
                         preferred_element_type=jnp.float32) + biv1[...])
    return jnp.tanh(jnp.dot(h, Wiv2[...],
                            preferred_element_type=jnp.float32) + biv2[...])

  def fc_sig(x, W1, b1, W2, b2):
    h = jax.nn.relu(jnp.dot(x, W1[...],
                            preferred_element_type=jnp.float32) + b1[...])
    lg = jnp.sum(h * W2[...], axis=-1, keepdims=True) + b2[...]
    return jax.nn.sigmoid(lg)

  item_emb = jnp.dot(it_raw[...], Wti[...],
                     preferred_element_type=jnp.float32) + bti[...]
  query_rep = attn_pool(s_raw[...], src_i[...], L, Wtq, btq, Wsq, bsq, qsq)
  click_rep = attn_pool(c_raw[...], clk_i[...], L, Wti, bti, Wsc, bsc, qsc)
  browse_rep = attn_pool(b_raw[...], brw_i[...], L, Wti, bti, Wbi, bbi, qbi)

  iv_feats = iv_pool(s_raw[...], src_i[...], L)
  d1 = iv_feats - browse_rep
  s1_part = jnp.sum(d1 * d1) * (inv_b / D)

  uw = fc_sig(jnp.concatenate([iv_feats, browse_rep], axis=-1),
              WuA1, buA1, WuA2, buA2)
  iv_user = uw * iv_feats + (1.0 - uw) * browse_rep

  def u_branch(x):
    u = jnp.tanh(jnp.dot(x, Wur[...], preferred_element_type=jnp.float32)
                 + bur[...])
    hu = jnp.tanh(jnp.dot(u, Wua[...], preferred_element_type=jnp.float32)
                  + bua[...])
    su = jnp.dot(hu, qua[...].reshape(100, 1),
                 preferred_element_type=jnp.float32)
    return u, su

  u0, su0 = u_branch(iv_user)
  u1, su1 = u_branch(query_rep)
  u2, su2 = u_branch(click_rep)
  su = jnp.concatenate([su0, su1, su2], axis=-1)
  au = jax.nn.softmax(su, axis=-1)
  user_rep = (au[:, 0:1] * u0 + au[:, 1:2] * u1 + au[:, 2:3] * u2)

  iv_item = iv_pool(iq_raw[...], iq_i[...], LQ)
  d2 = iv_item - item_emb
  s1i_part = jnp.sum(d2 * d2) * (inv_b / D)

  iw = fc_sig(jnp.concatenate([iv_item, item_emb], axis=-1),
              WiA1, biA1, WiA2, biA2)
  item_rep0 = iw * iv_item + (1.0 - iw) * item_emb
  item_rep = jnp.tanh(jnp.dot(item_rep0, Wir[...],
                              preferred_element_type=jnp.float32) + bir[...])

  logits = jnp.sum(item_rep * user_rep, axis=-1, keepdims=True)
  prob = jnp.clip(jax.nn.sigmoid(logits), 1e-7, 1.0 - 1e-7)
  y = lbl[...]
  bce_part = jnp.sum(-(y * jnp.log(prob) + (1.0 - y) * jnp.log(1.0 - prob))
                     ) * inv_b

  @pl.when(pid == 0)
  def _():
    o_bce[...] = jnp.zeros_like(o_bce)
    o_s1[...] = jnp.zeros_like(o_s1)
    o_s1i[...] = jnp.zeros_like(o_s1i)

  o_bce[...] += bce_part
  o_s1[...] += s1_part
  o_s1i[...] += s1i_part


def _dense(nb, *args):
  bb = 128
  grid = nb // bb

  def full(x):
    return pl.BlockSpec(x.shape, lambda i: (0,) * x.ndim)

  def rows(x):
    blk = x.shape[0] // grid
    return pl.BlockSpec((blk,) + x.shape[1:],
                        lambda i: (i,) + (0,) * (x.ndim - 1))

  weights = args[10:]
  in_specs = [rows(a) for a in args[:10]] + [full(w) for w in weights]
  out_spec = pl.BlockSpec((1, 1), lambda i: (0, 0))
  return pl.pallas_call(
      _dense_body,
      grid=(grid,),
      in_specs=in_specs,
      out_specs=(out_spec, out_spec, out_spec),
      out_shape=tuple(jax.ShapeDtypeStruct((1, 1), jnp.float32)
                      for _ in range(3)),
  )(*args)


def kernel(browse_item, src_qry, search_click, item, item_qry, labels,
           item_table, qry_table, Wti, bti, Wtq, btq, Wsq, bsq, qsq,
           Wsc, bsc, qsc, Wbi, bbi, qbi, Wir, bir, Wur, bur, Wua, bua, qua,
           Wiv1, biv1, Wiv2, biv2, WuA1, buA1, WuA2, buA2,
           WiA1, biA1, WiA2, biA2):
  r1 = lambda v: v.reshape(1, -1)
  weights = (
      Wti, r1(bti), Wtq, r1(btq),
      Wsq, r1(bsq), r1(qsq), Wsc, r1(bsc), r1(qsc), Wbi, r1(bbi), r1(qbi),
      Wir, r1(bir), Wur, r1(bur), Wua, r1(bua), r1(qua),
      Wiv1, r1(biv1), Wiv2, r1(biv2),
      WuA1, r1(buA1), WuA2.reshape(1, -1), r1(buA2),
      WiA1, r1(biA1), WiA2.reshape(1, -1), r1(biA2),
  )

  H = B // 2
  parts = []
  for h in range(2):
    sl = slice(h * H, (h + 1) * H)
    b_raw, c_raw, it_raw = _sc_gather_multi(
        item_table,
        [browse_item[sl].reshape(-1, _CH),
         search_click[sl].reshape(-1, _CH), item[sl].reshape(-1, _CH)])
    s_raw, iq_raw = _sc_gather_multi(
        qry_table,
        [src_qry[sl].reshape(-1, _CH), item_qry[sl].reshape(-1, _CH)])
    parts.append(_dense(
        H,
        s_raw, c_raw, b_raw, it_raw, iq_raw,
        src_qry[sl], search_click[sl], browse_item[sl], item_qry[sl],
        labels[sl].reshape(H, 1), *weights))
  return tuple(parts[0][j][0, 0] + parts[1][j][0, 0] for j in range(3))


# speedup vs baseline: 4.2622x; 1.0115x over previous
import functools

import jax
import jax.numpy as jnp
from jax import lax
from jax.experimental import pallas as pl
from jax.experimental.pallas import tpu as pltpu
from jax.experimental.pallas import tpu_sc as plsc

B = 4096
L = 50
LQ = 5
D = 64
DENSE = 128

_NC = 2
_NS = 16
_NW = _NC * _NS
_CH = 128


_NBUF = 4


def _sc_gather_multi(table, idxs):
  d = table.shape[1]
  seg_ch = []
  for ix in idxs:
    n_ch = ix.shape[0] // _NW
    assert n_ch * _NW == ix.shape[0]
    seg_ch.append(n_ch)
  tot_ch = sum(seg_ch)

  mesh = plsc.VectorSubcoreMesh(core_axis_name="c", subcore_axis_name="s")

  @functools.partial(
      pl.kernel,
      mesh=mesh,
      out_type=tuple(
          jax.ShapeDtypeStruct((ix.shape[0] * _CH, d), jnp.float32)
          for ix in idxs),
      scratch_types=[
          pltpu.VMEM((tot_ch, _CH), jnp.int32),
          pltpu.VMEM((_NBUF, _CH, d), jnp.float32),
      ] + [pltpu.SemaphoreType.DMA] * _NBUF,
      compiler_params=pltpu.CompilerParams(use_tc_tiling_on_sc=False),
  )
  def k(table_hbm, *refs):
    idx_hbms = refs[:len(idxs)]
    out_hbms = refs[len(idxs):2 * len(idxs)]
    idx_v = refs[2 * len(idxs)]
    rows_v = refs[2 * len(idxs) + 1]
    sems = refs[2 * len(idxs) + 2:]
    wid = lax.axis_index("s") * _NC + lax.axis_index("c")

    soff = 0
    for s, n_ch in enumerate(seg_ch):
      pltpu.sync_copy(idx_hbms[s].at[pl.ds(wid * n_ch, n_ch)],
                      idx_v.at[pl.ds(soff, n_ch)])
      soff += n_ch

    soff = 0
    for s, n_ch in enumerate(seg_ch):
      out = out_hbms[s]
      rbase = wid * n_ch
      n_grp = n_ch // _NBUF

      def body(g, carry, soff=soff, out=out, rbase=rbase):
        handles = []
        for b in range(_NBUF):
          i = g * _NBUF + b
          handles.append(
              pltpu.async_copy(table_hbm.at[idx_v.at[soff + i]],
                               rows_v.at[b], sems[b]))
        for b in range(_NBUF):
          i = g * _NBUF + b
          handles[b].wait()
          pltpu.sync_copy(rows_v.at[b],
                          out.at[pl.ds((rbase + i) * _CH, _CH)])
        return carry

      if n_grp > 0:
        lax.fori_loop(0, n_grp, body, 0, unroll=False)
      for i in range(n_grp * _NBUF, n_ch):
        pltpu.async_copy(table_hbm.at[idx_v.at[soff + i]], rows_v.at[0],
                         sems[0]).wait()
        pltpu.sync_copy(rows_v.at[0], out.at[pl.ds((rbase + i) * _CH, _CH)])
      soff += n_ch

  return k(table, *idxs)


def _dense_body(
    s_raw, c_raw, b_raw, it_raw, iq_raw,
    src_i, clk_i, brw_i, iq_i, lbl,
    Wti, bti, Wtq, btq,
    Wsq, bsq, qsq, Wsc, bsc, qsc, Wbi, bbi, qbi,
    Wir, bir, Wur, bur, Wua, bua, qua,
    Wiv1, biv1, Wiv2, biv2,
    WuA1, buA1, WuA2, buA2, WiA1, biA1, WiA2, biA2,
    o_bce, o_s1, o_s1i,
):
  pid = pl.program_id(0)
  bb = src_i.shape[0]
  inv_b = jnp.float32(1.0 / B)

  def attn_pool(x2, idx, ll, Wt, bt, W, b, q):
    A = jnp.dot(Wt[...], W[...], preferred_element_type=jnp.float32)
    c = jnp.dot(bt[...], W[...], preferred_element_type=jnp.float32) + b[...]
    h = jnp.tanh(jnp.dot(x2, A, preferred_element_type=jnp.float32) + c)
    s = jnp.dot(h, q[...].reshape(DENSE, 1),
                preferred_element_type=jnp.float32).reshape(bb, ll)
    s = jnp.where(idx == 0, jnp.float32(-1e9), s)
    a = jax.nn.softmax(s, axis=-1)
    pooled = jnp.sum(a[:, :, None] * x2.reshape(bb, ll, D), axis=1)
    return jnp.dot(pooled, Wt[...], preferred_element_type=jnp.float32) + bt[...]

  def iv_pool(x2, idx, ll):
    m = (idx != 0).astype(jnp.float32)
    pooled = jnp.sum(m[:, :, None] * x2.reshape(bb, ll, D), axis=1)
    cnt = jnp.maximum(jnp.sum(m, axis=1, keepdims=True), 1.0)
    pooled = pooled / cnt
    h = jnp.tanh(jnp.dot(pooled, Wiv1[...],
                         preferred_element_type=jnp.float32) + biv1[...])
    return jnp.tanh(jnp.dot(h, Wiv2[...],
                            preferred_element_type=jnp.float32) + biv2[...])

  def fc_sig(x, W1, b1, W2, b2):
    h = jax.nn.relu(jnp.dot(x, W1[...],
                            preferred_element_type=jnp.float32) + b1[...])
    lg = jnp.sum(h * W2[...], axis=-1, keepdims=True) + b2[...]
    return jax.nn.sigmoid(lg)

  item_emb = jnp.dot(it_raw[...], Wti[...],
                     preferred_element_type=jnp.float32) + bti[...]
  query_rep = attn_pool(s_raw[...], src_i[...], L, Wtq, btq, Wsq, bsq, qsq)
  click_rep = attn_pool(c_raw[...], clk_i[...], L, Wti, bti, Wsc, bsc, qsc)
  browse_rep = attn_pool(b_raw[...], brw_i[...], L, Wti, bti, Wbi, bbi, qbi)

  iv_feats = iv_pool(s_raw[...], src_i[...], L)
  d1 = iv_feats - browse_rep
  s1_part = jnp.sum(d1 * d1) * (inv_b / D)

  uw = fc_sig(jnp.concatenate([iv_feats, browse_rep], axis=-1),
              WuA1, buA1, WuA2, buA2)
  iv_user = uw * iv_feats + (1.0 - uw) * browse_rep

  def u_branch(x):
    u = jnp.tanh(jnp.dot(x, Wur[...], preferred_element_type=jnp.float32)
                 + bur[...])
    hu = jnp.tanh(jnp.dot(u, Wua[...], preferred_element_type=jnp.float32)
                  + bua[...])
    su = jnp.dot(hu, qua[...].reshape(100, 1),
                 preferred_element_type=jnp.float32)
    return u, su

  u0, su0 = u_branch(iv_user)
  u1, su1 = u_branch(query_rep)
  u2, su2 = u_branch(click_rep)
  su = jnp.concatenate([su0, su1, su2], axis=-1)
  au = jax.nn.softmax(su, axis=-1)
  user_rep = (au[:, 0:1] * u0 + au[:, 1:2] * u1 + au[:, 2:3] * u2)

  iv_item = iv_pool(iq_raw[...], iq_i[...], LQ)
  d2 = iv_item - item_emb
  s1i_part = jnp.sum(d2 * d2) * (inv_b / D)

  iw = fc_sig(jnp.concatenate([iv_item, item_emb], axis=-1),
              WiA1, biA1, WiA2, biA2)
  item_rep0 = iw * iv_item + (1.0 - iw) * item_emb
  item_rep = jnp.tanh(jnp.dot(item_rep0, Wir[...],
                              preferred_element_type=jnp.float32) + bir[...])

  logits = jnp.sum(item_rep * user_rep, axis=-1, keepdims=True)
  prob = jnp.clip(jax.nn.sigmoid(logits), 1e-7, 1.0 - 1e-7)
  y = lbl[...]
  bce_part = jnp.sum(-(y * jnp.log(prob) + (1.0 - y) * jnp.log(1.0 - prob))
                     ) * inv_b

  @pl.when(pid == 0)
  def _():
    o_bce[...] = jnp.zeros_like(o_bce)
    o_s1[...] = jnp.zeros_like(o_s1)
    o_s1i[...] = jnp.zeros_like(o_s1i)

  o_bce[...] += bce_part
  o_s1[...] += s1_part
  o_s1i[...] += s1i_part


def _dense(interpret, *args):
  bb = 128
  grid = B // bb

  def full(x):
    return pl.BlockSpec(x.shape, lambda i: (0,) * x.ndim)

  def rows(x):
    blk = x.shape[0] // grid
    return pl.BlockSpec((blk,) + x.shape[1:],
                        lambda i: (i,) + (0,) * (x.ndim - 1))

  weights = args[10:]
  in_specs = [rows(a) for a in args[:10]] + [full(w) for w in weights]
  out_spec = pl.BlockSpec((1, 1), lambda i: (0, 0))
  return pl.pallas_call(
      _dense_body,
      grid=(grid,),
      in_specs=in_specs,
      out_specs=(out_spec, out_spec, out_spec),
      out_shape=tuple(jax.ShapeDtypeStruct((1, 1), jnp.float32)
                      for _ in range(3)),
      interpret=interpret,
  )(*args)


def kernel(browse_item, src_qry, search_click, item, item_qry, labels,
           item_table, qry_table, Wti, bti, Wtq, btq, Wsq, bsq, qsq,
           Wsc, bsc, qsc, Wbi, bbi, qbi, Wir, bir, Wur, bur, Wua, bua, qua,
           Wiv1, biv1, Wiv2, biv2, WuA1, buA1, WuA2, buA2,
           WiA1, biA1, WiA2, biA2):
  b_raw, c_raw, it_raw = _sc_gather_multi(
      item_table,
      [browse_item.reshape(-1, _CH), search_click.reshape(-1, _CH),
       item.reshape(-1, _CH)])
  s_raw, iq_raw = _sc_gather_multi(
      qry_table,
      [src_qry.reshape(-1, _CH), item_qry.reshape(-1, _CH)])

  r1 = lambda v: v.reshape(1, -1)
  o_bce, o_s1, o_s1i = _dense(
      False,
      s_raw, c_raw, b_raw, it_raw, iq_raw,
      src_qry, search_click, browse_item, item_qry, labels.reshape(B, 1),
      Wti, r1(bti), Wtq, r1(btq),
      Wsq, r1(bsq), r1(qsq), Wsc, r1(bsc), r1(qsc), Wbi, r1(bbi), r1(qbi),
      Wir, r1(bir), Wur, r1(bur), Wua, r1(bua), r1(qua),
      Wiv1, r1(biv1), Wiv2, r1(biv2),
      r1(WuA1) if WuA1.ndim == 1 else WuA1, r1(buA1),
      WuA2.reshape(1, -1), r1(buA2),
      WiA1, r1(biA1), WiA2.reshape(1, -1), r1(biA2),
  )
  return (o_bce[0, 0], o_s1[0, 0], o_s1i[0, 0])


# speedup vs baseline: 4.2913x; 1.0068x over previous
import functools

import jax
import jax.numpy as jnp
from jax import lax
from jax.experimental import pallas as pl
from jax.experimental.pallas import tpu as pltpu
from jax.experimental.pallas import tpu_sc as plsc

B = 4096
L = 50
LQ = 5
D = 64
DENSE = 128

_NC = 2
_NS = 16
_NW = _NC * _NS
_CH = 128


_NBUF = 2


def _sc_gather_multi(table, idxs):
  d = table.shape[1]
  seg_ch = []
  for ix in idxs:
    n_ch = ix.shape[0] // _NW
    assert n_ch * _NW == ix.shape[0]
    seg_ch.append(n_ch)
  tot_ch = sum(seg_ch)

  mesh = plsc.VectorSubcoreMesh(core_axis_name="c", subcore_axis_name="s")

  @functools.partial(
      pl.kernel,
      mesh=mesh,
      out_type=tuple(
          jax.ShapeDtypeStruct((ix.shape[0] * _CH, d), jnp.float32)
          for ix in idxs),
      scratch_types=[
          pltpu.VMEM((tot_ch, _CH), jnp.int32),
          pltpu.VMEM((_NBUF, _CH, d), jnp.float32),
      ] + [pltpu.SemaphoreType.DMA] * _NBUF,
      compiler_params=pltpu.CompilerParams(use_tc_tiling_on_sc=False),
  )
  def k(table_hbm, *refs):
    idx_hbms = refs[:len(idxs)]
    out_hbms = refs[len(idxs):2 * len(idxs)]
    idx_v = refs[2 * len(idxs)]
    rows_v = refs[2 * len(idxs) + 1]
    sems = refs[2 * len(idxs) + 2:]
    wid = lax.axis_index("s") * _NC + lax.axis_index("c")

    soff = 0
    for s, n_ch in enumerate(seg_ch):
      pltpu.sync_copy(idx_hbms[s].at[pl.ds(wid * n_ch, n_ch)],
                      idx_v.at[pl.ds(soff, n_ch)])
      soff += n_ch

    soff = 0
    for s, n_ch in enumerate(seg_ch):
      out = out_hbms[s]
      rbase = wid * n_ch
      n_grp = n_ch // _NBUF

      def body(g, carry, soff=soff, out=out, rbase=rbase):
        handles = []
        for b in range(_NBUF):
          i = g * _NBUF + b
          handles.append(
              pltpu.async_copy(table_hbm.at[idx_v.at[soff + i]],
                               rows_v.at[b], sems[b]))
        for b in range(_NBUF):
          i = g * _NBUF + b
          handles[b].wait()
          pltpu.sync_copy(rows_v.at[b],
                          out.at[pl.ds((rbase + i) * _CH, _CH)])
        return carry

      if n_grp > 0:
        lax.fori_loop(0, n_grp, body, 0, unroll=False)
      for i in range(n_grp * _NBUF, n_ch):
        pltpu.async_copy(table_hbm.at[idx_v.at[soff + i]], rows_v.at[0],
                         sems[0]).wait()
        pltpu.sync_copy(rows_v.at[0], out.at[pl.ds((rbase + i) * _CH, _CH)])
      soff += n_ch

  return k(table, *idxs)


def _dense_body(
    s_raw, c_raw, b_raw, it_raw, iq_raw,
    src_i, clk_i, brw_i, iq_i, lbl,
    Wti, bti, Wtq, btq,
    Wsq, bsq, qsq, Wsc, bsc, qsc, Wbi, bbi, qbi,
    Wir, bir, Wur, bur, Wua, bua, qua,
    Wiv1, biv1, Wiv2, biv2,
    WuA1, buA1, WuA2, buA2, WiA1, biA1, WiA2, biA2,
    o_bce, o_s1, o_s1i,
):
  pid = pl.program_id(0)
  bb = src_i.shape[0]
  inv_b = jnp.float32(1.0 / B)

  def attn_pool(x2, idx, ll, Wt, bt, W, b, q):
    A = jnp.dot(Wt[...], W[...], preferred_element_type=jnp.float32)
    c = jnp.dot(bt[...], W[...], preferred_element_type=jnp.float32) + b[...]
    h = jnp.tanh(jnp.dot(x2, A, preferred_element_type=jnp.float32) + c)
    s = jnp.dot(h, q[...].reshape(DENSE, 1),
                preferred_element_type=jnp.float32).reshape(bb, ll)
    s = jnp.where(idx == 0, jnp.float32(-1e9), s)
    a = jax.nn.softmax(s, axis=-1)
    pooled = jnp.sum(a[:, :, None] * x2.reshape(bb, ll, D), axis=1)
    return jnp.dot(pooled, Wt[...], preferred_element_type=jnp.float32) + bt[...]

  def iv_pool(x2, idx, ll):
    m = (idx != 0).astype(jnp.float32)
    pooled = jnp.sum(m[:, :, None] * x2.reshape(bb, ll, D), axis=1)
    cnt = jnp.maximum(jnp.sum(m, axis=1, keepdims=True), 1.0)
    pooled = pooled / cnt
    h = jnp.tanh(jnp.dot(pooled, Wiv1[...],
                         preferred_element_type=jnp.float32) + biv1[...])
    return jnp.tanh(jnp.dot(h, Wiv2[...],
                            preferred_element_type=jnp.float32) + biv2[...])

  def fc_sig(x, W1, b1, W2, b2):
    h = jax.nn.relu(jnp.dot(x, W1[...],
                            preferred_element_type=jnp.float32) + b1[...])
    lg = jnp.sum(h * W2[...], axis=-1, keepdims=True) + b2[...]
    return jax.nn.sigmoid(lg)

  item_emb = jnp.dot(it_raw[...], Wti[...],
                     preferred_element_type=jnp.float32) + bti[...]
  query_rep = attn_pool(s_raw[...], src_i[...], L, Wtq, btq, Wsq, bsq, qsq)
  click_rep = attn_pool(c_raw[...], clk_i[...], L, Wti, bti, Wsc, bsc, qsc)
  browse_rep = attn_pool(b_raw[...], brw_i[...], L, Wti, bti, Wbi, bbi, qbi)

  iv_feats = iv_pool(s_raw[...], src_i[...], L)
  d1 = iv_feats - browse_rep
  s1_part = jnp.sum(d1 * d1) * (inv_b / D)

  uw = fc_sig(jnp.concatenate([iv_feats, browse_rep], axis=-1),
              WuA1, buA1, WuA2, buA2)
  iv_user = uw * iv_feats + (1.0 - uw) * browse_rep

  def u_branch(x):
    u = jnp.tanh(jnp.dot(x, Wur[...], preferred_element_type=jnp.float32)
                 + bur[...])
    hu = jnp.tanh(jnp.dot(u, Wua[...], preferred_element_type=jnp.float32)
                  + bua[...])
    su = jnp.dot(hu, qua[...].reshape(100, 1),
                 preferred_element_type=jnp.float32)
    return u, su

  u0, su0 = u_branch(iv_user)
  u1, su1 = u_branch(query_rep)
  u2, su2 = u_branch(click_rep)
  su = jnp.concatenate([su0, su1, su2], axis=-1)
  au = jax.nn.softmax(su, axis=-1)
  user_rep = (au[:, 0:1] * u0 + au[:, 1:2] * u1 + au[:, 2:3] * u2)

  iv_item = iv_pool(iq_raw[...], iq_i[...], LQ)
  d2 = iv_item - item_emb
  s1i_part = jnp.sum(d2 * d2) * (inv_b / D)

  iw = fc_sig(jnp.concatenate([iv_item, item_emb], axis=-1),
              WiA1, biA1, WiA2, biA2)
  item_rep0 = iw * iv_item + (1.0 - iw) * item_emb
  item_rep = jnp.tanh(jnp.dot(item_rep0, Wir[...],
                              preferred_element_type=jnp.float32) + bir[...])

  logits = jnp.sum(item_rep * user_rep, axis=-1, keepdims=True)
  prob = jnp.clip(jax.nn.sigmoid(logits), 1e-7, 1.0 - 1e-7)
  y = lbl[...]
  bce_part = jnp.sum(-(y * jnp.log(prob) + (1.0 - y) * jnp.log(1.0 - prob))
                     ) * inv_b

  @pl.when(pid == 0)
  def _():
    o_bce[...] = jnp.zeros_like(o_bce)
    o_s1[...] = jnp.zeros_like(o_s1)
    o_s1i[...] = jnp.zeros_like(o_s1i)

  o_bce[...] += bce_part
  o_s1[...] += s1_part
  o_s1i[...] += s1i_part


def _dense(interpret, *args):
  bb = 128
  grid = B // bb

  def full(x):
    return pl.BlockSpec(x.shape, lambda i: (0,) * x.ndim)

  def rows(x):
    blk = x.shape[0] // grid
    return pl.BlockSpec((blk,) + x.shape[1:],
                        lambda i: (i,) + (0,) * (x.ndim - 1))

  weights = args[10:]
  in_specs = [rows(a) for a in args[:10]] + [full(w) for w in weights]
  out_spec = pl.BlockSpec((1, 1), lambda i: (0, 0))
  return pl.pallas_call(
      _dense_body,
      grid=(grid,),
      in_specs=in_specs,
      out_specs=(out_spec, out_spec, out_spec),
      out_shape=tuple(jax.ShapeDtypeStruct((1, 1), jnp.float32)
                      for _ in range(3)),
      interpret=interpret,
  )(*args)


def kernel(browse_item, src_qry, search_click, item, item_qry, labels,
           item_table, qry_table, Wti, bti, Wtq, btq, Wsq, bsq, qsq,
           Wsc, bsc, qsc, Wbi, bbi, qbi, Wir, bir, Wur, bur, Wua, bua, qua,
           Wiv1, biv1, Wiv2, biv2, WuA1, buA1, WuA2, buA2,
           WiA1, biA1, WiA2, biA2):
  b_raw, c_raw, it_raw = _sc_gather_multi(
      item_table,
      [browse_item.reshape(-1, _CH), search_click.reshape(-1, _CH),
       item.reshape(-1, _CH)])
  s_raw, iq_raw = _sc_gather_multi(
      qry_table,
      [src_qry.reshape(-1, _CH), item_qry.reshape(-1, _CH)])

  r1 = lambda v: v.reshape(1, -1)
  o_bce, o_s1, o_s1i = _dense(
      False,
      s_raw, c_raw, b_raw, it_raw, iq_raw,
      src_qry, search_click, browse_item, item_qry, labels.reshape(B, 1),
      Wti, r1(bti), Wtq, r1(btq),
      Wsq, r1(bsq), r1(qsq), Wsc, r1(bsc), r1(qsc), Wbi, r1(bbi), r1(qbi),
      Wir, r1(bir), Wur, r1(bur), Wua, r1(bua), r1(qua),
      Wiv1, r1(biv1), Wiv2, r1(biv2),
      r1(WuA1) if WuA1.ndim == 1 else WuA1, r1(buA1),
      WuA2.reshape(1, -1), r1(buA2),
      WiA1, r1(biA1), WiA2.reshape(1, -1), r1(biA2),
  )
  return (o_bce[0, 0], o_s1[0, 0], o_s1i[0, 0])
